# baseline (speedup 1.0000x reference)
import functools

import jax
import jax.numpy as jnp
from jax import lax
from jax.experimental import pallas as pl
from jax.experimental.pallas import tpu as pltpu
from jax.experimental.pallas import tpu_sc as plsc

DIM = 768
BATCH = 64
SEQ = 576
ROWS = BATCH * SEQ
TAB = 64

NUM_CORES = 2
NUM_SUBCORES = 16
NUM_WORKERS = NUM_CORES * NUM_SUBCORES
ROWS_PER_WORKER = ROWS // NUM_WORKERS
LANES = 16
IDX_STEPS = ROWS_PER_WORKER // LANES
CHUNK = 48
NUM_CHUNKS = ROWS_PER_WORKER // CHUNK
NUM_TRIPLES = NUM_CHUNKS // 3


def _sum_table_tc(h_ref, w_ref, sum_ref):
    h = h_ref[...]
    w = w_ref[...]
    sum_ref[...] = h[:, None, :] + w[None, :, :]


def _build_sum_table(h_table, w_table):
    return pl.pallas_call(
        _sum_table_tc,
        out_shape=jax.ShapeDtypeStruct((TAB, TAB, DIM), jnp.float32),
    )(h_table, w_table)


_MESH = plsc.VectorSubcoreMesh(core_axis_name="c", subcore_axis_name="s")


@functools.partial(
    pl.kernel,
    out_type=jax.ShapeDtypeStruct((ROWS, DIM), jnp.float32),
    mesh=_MESH,
    scratch_types=[
        pltpu.VMEM((ROWS_PER_WORKER,), jnp.int32),
        pltpu.VMEM((ROWS_PER_WORKER,), jnp.int32),
        pltpu.VMEM((CHUNK, DIM), jnp.float32),
        pltpu.VMEM((CHUNK, DIM), jnp.float32),
        pltpu.VMEM((CHUNK, DIM), jnp.float32),
        pltpu.SemaphoreType.DMA,
        pltpu.SemaphoreType.DMA,
        pltpu.SemaphoreType.DMA,
        pltpu.SemaphoreType.DMA,
        pltpu.SemaphoreType.DMA,
        pltpu.SemaphoreType.DMA,
    ],
)
def _gather_sc(h_idx_hbm, w_idx_hbm, sum_tab_hbm, out_hbm,
               idx_v, widx_v, buf_0, buf_1, buf_2,
               gs_0, gs_1, gs_2, os_0, os_1, os_2):
    wid = lax.axis_index("s") * NUM_CORES + lax.axis_index("c")
    base = wid * ROWS_PER_WORKER
    pltpu.sync_copy(h_idx_hbm.at[pl.ds(base, ROWS_PER_WORKER)], idx_v)
    pltpu.sync_copy(w_idx_hbm.at[pl.ds(base, ROWS_PER_WORKER)], widx_v)

    def fuse_body(k, carry):
        sl = pl.ds(k * LANES, LANES)
        idx_v[sl] = idx_v[sl] * TAB + widx_v[sl]
        return carry

    lax.fori_loop(0, IDX_STEPS, fuse_body, 0)

    bufs = (buf_0, buf_1, buf_2)
    gsems = (gs_0, gs_1, gs_2)
    osems = (os_0, os_1, os_2)

    def gather(chunk, slot):
        pltpu.async_copy(
            sum_tab_hbm.at[idx_v.at[pl.ds(chunk * CHUNK, CHUNK)]],
            bufs[slot], gsems[slot])

    def wait_gather(chunk, slot):
        pltpu.make_async_copy(
            sum_tab_hbm.at[idx_v.at[pl.ds(chunk * CHUNK, CHUNK)]],
            bufs[slot], gsems[slot]).wait()

    def writeback(chunk, slot):
        pltpu.async_copy(
            bufs[slot], out_hbm.at[pl.ds(base + chunk * CHUNK, CHUNK)],
            osems[slot])

    def wait_writeback(chunk, slot):
        pltpu.make_async_copy(
            bufs[slot], out_hbm.at[pl.ds(base + chunk * CHUNK, CHUNK)],
            osems[slot]).wait()

    gather(0, 0)
    gather(1, 1)

    def triple_body(t, carry):
        for s in range(3):
            i = 3 * t + s

            def step(first, last):
                wait_gather(i, s)
                writeback(i, s)
                nxt = (s + 2) % 3

                @pl.when(jnp.logical_not(first))
                def _():
                    wait_writeback(i - 1, nxt)

                @pl.when(jnp.logical_not(last))
                def _():
                    gather(i + 2, nxt)

            if s == 0:
                step(t == 0, jnp.bool_(False))
            else:
                step(jnp.bool_(False), t == NUM_TRIPLES - 1)
        return carry

    lax.fori_loop(0, NUM_TRIPLES, triple_body, 0)
    wait_writeback(NUM_CHUNKS - 1, 2)


def kernel(position_ids, h_table, w_table):
    h_idx = position_ids[..., 0].reshape(ROWS).astype(jnp.int32)
    w_idx = position_ids[..., 1].reshape(ROWS).astype(jnp.int32)
    sum_tab = _build_sum_table(h_table, w_table).reshape(TAB * TAB, DIM)
    out = _gather_sc(h_idx, w_idx, sum_tab)
    return out.reshape(BATCH, SEQ, DIM)


# speedup vs baseline: 1.0197x; 1.0197x over previous
import functools

import jax
import jax.numpy as jnp
from jax import lax
from jax.experimental import pallas as pl
from jax.experimental.pallas import tpu as pltpu
from jax.experimental.pallas import tpu_sc as plsc

DIM = 768
BATCH = 64
SEQ = 576
ROWS = BATCH * SEQ
TAB = 64

NUM_CORES = 2
NUM_SUBCORES = 16
NUM_WORKERS = NUM_CORES * NUM_SUBCORES
ROWS_PER_WORKER = ROWS // NUM_WORKERS
LANES = 16
IDX_STEPS = ROWS_PER_WORKER // LANES
CHUNK = 48
NUM_CHUNKS = ROWS_PER_WORKER // CHUNK
NUM_TRIPLES = NUM_CHUNKS // 3


def _sum_table_tc(h_ref, w_ref, sum_ref):
    h = h_ref[...]
    w = w_ref[...]
    sum_ref[...] = h[:, None, :] + w[None, :, :]


def _build_sum_table(h_table, w_table):
    return pl.pallas_call(
        _sum_table_tc,
        out_shape=jax.ShapeDtypeStruct((TAB, TAB, DIM), jnp.float32),
    )(h_table, w_table)


_MESH = plsc.VectorSubcoreMesh(core_axis_name="c", subcore_axis_name="s")


@functools.partial(
    pl.kernel,
    out_type=jax.ShapeDtypeStruct((ROWS, DIM), jnp.float32),
    mesh=_MESH,
    scratch_types=[
        pltpu.VMEM((ROWS_PER_WORKER,), jnp.int32),
        pltpu.VMEM((CHUNK, DIM), jnp.float32),
        pltpu.VMEM((CHUNK, DIM), jnp.float32),
        pltpu.VMEM((CHUNK, DIM), jnp.float32),
        pltpu.SemaphoreType.DMA,
        pltpu.SemaphoreType.DMA,
        pltpu.SemaphoreType.DMA,
        pltpu.SemaphoreType.DMA,
        pltpu.SemaphoreType.DMA,
        pltpu.SemaphoreType.DMA,
    ],
)
def _gather_sc(fidx_hbm, sum_tab_hbm, out_hbm,
               idx_v, buf_0, buf_1, buf_2,
               gs_0, gs_1, gs_2, os_0, os_1, os_2):
    wid = lax.axis_index("s") * NUM_CORES + lax.axis_index("c")
    base = wid * ROWS_PER_WORKER
    pltpu.sync_copy(fidx_hbm.at[pl.ds(base, ROWS_PER_WORKER)], idx_v)

    bufs = (buf_0, buf_1, buf_2)
    gsems = (gs_0, gs_1, gs_2)
    osems = (os_0, os_1, os_2)

    def gather(chunk, slot):
        pltpu.async_copy(
            sum_tab_hbm.at[idx_v.at[pl.ds(chunk * CHUNK, CHUNK)]],
            bufs[slot], gsems[slot])

    def wait_gather(chunk, slot):
        pltpu.make_async_copy(
            sum_tab_hbm.at[idx_v.at[pl.ds(chunk * CHUNK, CHUNK)]],
            bufs[slot], gsems[slot]).wait()

    def writeback(chunk, slot):
        pltpu.async_copy(
            bufs[slot], out_hbm.at[pl.ds(base + chunk * CHUNK, CHUNK)],
            osems[slot])

    def wait_writeback(chunk, slot):
        pltpu.make_async_copy(
            bufs[slot], out_hbm.at[pl.ds(base + chunk * CHUNK, CHUNK)],
            osems[slot]).wait()

    gather(0, 0)
    gather(1, 1)

    def triple_body(t, carry):
        for s in range(3):
            i = 3 * t + s

            def step(first, last):
                wait_gather(i, s)
                writeback(i, s)
                nxt = (s + 2) % 3

                @pl.when(jnp.logical_not(first))
                def _():
                    wait_writeback(i - 1, nxt)

                @pl.when(jnp.logical_not(last))
                def _():
                    gather(i + 2, nxt)

            if s == 0:
                step(t == 0, jnp.bool_(False))
            else:
                step(jnp.bool_(False), t == NUM_TRIPLES - 1)
        return carry

    lax.fori_loop(0, NUM_TRIPLES, triple_body, 0)
    wait_writeback(NUM_CHUNKS - 1, 2)


def kernel(position_ids, h_table, w_table):
    ids = position_ids.astype(jnp.int32)
    fidx = (ids[..., 0] * TAB + ids[..., 1]).reshape(ROWS)
    sum_tab = _build_sum_table(h_table, w_table).reshape(TAB * TAB, DIM)
    out = _gather_sc(fidx, sum_tab)
    return out.reshape(BATCH, SEQ, DIM)


# speedup vs baseline: 1.0650x; 1.0444x over previous
import functools

import jax
import jax.numpy as jnp
from jax import lax
from jax.experimental import pallas as pl
from jax.experimental.pallas import tpu as pltpu
from jax.experimental.pallas import tpu_sc as plsc

DIM = 768
BATCH = 64
SEQ = 576
ROWS = BATCH * SEQ
TAB = 64

NUM_CORES = 2
NUM_SUBCORES = 16
NUM_WORKERS = NUM_CORES * NUM_SUBCORES

SC_ROWS = 18432
TC_ROWS = ROWS - SC_ROWS

ROWS_PER_WORKER = SC_ROWS // NUM_WORKERS
CHUNK = 48
NUM_CHUNKS = ROWS_PER_WORKER // CHUNK
NUM_TRIPLES = NUM_CHUNKS // 3

TCB = 512
HEAD_BLOCKS = SC_ROWS // TCB
TC_BLOCKS = TC_ROWS // TCB


def _sum_table_tc(h_ref, w_ref, sum_ref, tab2_ref):
    h = h_ref[...]
    w = w_ref[...]
    sum_ref[...] = h[:, None, :] + w[None, :, :]
    tab2_ref[...] = jnp.concatenate([h, w], axis=0)


def _build_sum_table(h_table, w_table):
    return pl.pallas_call(
        _sum_table_tc,
        out_shape=[
            jax.ShapeDtypeStruct((TAB, TAB, DIM), jnp.float32),
            jax.ShapeDtypeStruct((2 * TAB, DIM), jnp.float32),
        ],
    )(h_table, w_table)


def _tail_tc(fidx_ref, tab2_ref, sc_ref, out_ref):
    del sc_ref
    k = fidx_ref[...][:, None]
    iota = lax.broadcasted_iota(jnp.int32, (TCB, TAB), 1)
    oh_h = ((k >> 6) == iota).astype(jnp.float32)
    oh_w = ((k & (TAB - 1)) == iota).astype(jnp.float32)
    oh = jnp.concatenate([oh_h, oh_w], axis=1)
    out_ref[...] = lax.dot_general(
        oh, tab2_ref[...], (((1,), (0,)), ((), ())),
        preferred_element_type=jnp.float32)


def _tail_lookup(fidx, tab2, sc_out):
    return pl.pallas_call(
        _tail_tc,
        grid=(TC_BLOCKS,),
        in_specs=[
            pl.BlockSpec((TCB,), lambda i: (HEAD_BLOCKS + i,)),
            pl.BlockSpec((2 * TAB, DIM), lambda i: (0, 0)),
            pl.BlockSpec(memory_space=pl.ANY),
        ],
        out_specs=pl.BlockSpec((TCB, DIM), lambda i: (HEAD_BLOCKS + i, 0)),
        out_shape=jax.ShapeDtypeStruct((ROWS, DIM), jnp.float32),
        input_output_aliases={2: 0},
    )(fidx, tab2, sc_out)


_MESH = plsc.VectorSubcoreMesh(core_axis_name="c", subcore_axis_name="s")


@functools.partial(
    pl.kernel,
    out_type=jax.ShapeDtypeStruct((ROWS, DIM), jnp.float32),
    mesh=_MESH,
    scratch_types=[
        pltpu.VMEM((ROWS_PER_WORKER,), jnp.int32),
        pltpu.VMEM((CHUNK, DIM), jnp.float32),
        pltpu.VMEM((CHUNK, DIM), jnp.float32),
        pltpu.VMEM((CHUNK, DIM), jnp.float32),
        pltpu.SemaphoreType.DMA,
        pltpu.SemaphoreType.DMA,
        pltpu.SemaphoreType.DMA,
        pltpu.SemaphoreType.DMA,
        pltpu.SemaphoreType.DMA,
        pltpu.SemaphoreType.DMA,
    ],
)
def _gather_sc(fidx_hbm, sum_tab_hbm, out_hbm,
               idx_v, buf_0, buf_1, buf_2,
               gs_0, gs_1, gs_2, os_0, os_1, os_2):
    wid = lax.axis_index("s") * NUM_CORES + lax.axis_index("c")
    base = wid * ROWS_PER_WORKER
    pltpu.sync_copy(fidx_hbm.at[pl.ds(base, ROWS_PER_WORKER)], idx_v)

    bufs = (buf_0, buf_1, buf_2)
    gsems = (gs_0, gs_1, gs_2)
    osems = (os_0, os_1, os_2)

    def gather(chunk, slot):
        pltpu.async_copy(
            sum_tab_hbm.at[idx_v.at[pl.ds(chunk * CHUNK, CHUNK)]],
            bufs[slot], gsems[slot])

    def wait_gather(chunk, slot):
        pltpu.make_async_copy(
            sum_tab_hbm.at[idx_v.at[pl.ds(chunk * CHUNK, CHUNK)]],
            bufs[slot], gsems[slot]).wait()

    def writeback(chunk, slot):
        pltpu.async_copy(
            bufs[slot], out_hbm.at[pl.ds(base + chunk * CHUNK, CHUNK)],
            osems[slot])

    def wait_writeback(chunk, slot):
        pltpu.make_async_copy(
            bufs[slot], out_hbm.at[pl.ds(base + chunk * CHUNK, CHUNK)],
            osems[slot]).wait()

    gather(0, 0)
    gather(1, 1)

    def triple_body(t, carry):
        for s in range(3):
            i = 3 * t + s

            def step(first, last):
                wait_gather(i, s)
                writeback(i, s)
                nxt = (s + 2) % 3

                @pl.when(jnp.logical_not(first))
                def _():
                    wait_writeback(i - 1, nxt)

                @pl.when(jnp.logical_not(last))
                def _():
                    gather(i + 2, nxt)

            if s == 0:
                step(t == 0, jnp.bool_(False))
            else:
                step(jnp.bool_(False), t == NUM_TRIPLES - 1)
        return carry

    lax.fori_loop(0, NUM_TRIPLES, triple_body, 0)
    wait_writeback(NUM_CHUNKS - 1, 2)


def kernel(position_ids, h_table, w_table):
    ids = position_ids.astype(jnp.int32)
    fidx = (ids[..., 0] * TAB + ids[..., 1]).reshape(ROWS)
    sum_tab, tab2 = _build_sum_table(h_table, w_table)
    sc_out = _gather_sc(fidx, sum_tab.reshape(TAB * TAB, DIM))
    out = _tail_lookup(fidx, tab2, sc_out)
    return out.reshape(BATCH, SEQ, DIM)
